# baseline (speedup 1.0000x reference)
import functools

import jax
import jax.numpy as jnp
from jax import lax
from jax.experimental import pallas as pl
from jax.experimental.pallas import tpu as pltpu
from jax.experimental.pallas import tpu_sc as plsc

NC = 2
NS = 16
NW = NC * NS
K = 80
CW = 16


def _mm_body(x_ref, w_ref, o_ref):
    o_ref[0] = jnp.dot(x_ref[...], w_ref[0],
                       preferred_element_type=jnp.float32)


def _tc_matmul_stacked(x, wstack):
    g, d, _ = wstack.shape
    n = x.shape[0]
    blk = 400
    return pl.pallas_call(
        _mm_body,
        grid=(g, n // blk),
        in_specs=[
            pl.BlockSpec((blk, d), lambda r, i: (i, 0)),
            pl.BlockSpec((1, d, d), lambda r, i: (r, 0, 0)),
        ],
        out_specs=pl.BlockSpec((1, blk, d), lambda r, i: (r, i, 0)),
        out_shape=jax.ShapeDtypeStruct((g, n, d), jnp.float32),
    )(x, wstack)


def _tc_index_prep(src2, dst2, et2, n):
    def body(s_ref, d_ref, t_ref, g_ref, w_ref):
        t = t_ref[...]
        g_ref[...] = t * n + s_ref[...]
        w_ref[...] = t * n + d_ref[...]

    shp = jax.ShapeDtypeStruct(src2.shape, jnp.int32)
    return pl.pallas_call(body, out_shape=(shp, shp))(src2, dst2, et2)


def _tc_inv(cpart):
    r, n = cpart.shape[1], cpart.shape[2]

    def body(c_ref, ic_ref, ic2_ref):
        cs = c_ref[0] + c_ref[1]
        ic_ref[...] = 1.0 / jnp.maximum(cs, 1.0)
        c2 = jnp.sum(cs, axis=0)
        ic2_ref[...] = (1.0 / jnp.maximum(c2, 1.0))[:, None]

    return pl.pallas_call(
        body,
        out_shape=(jax.ShapeDtypeStruct((r, n), jnp.float32),
                   jax.ShapeDtypeStruct((n, 1), jnp.float32)),
    )(cpart)


def _bn_lrelu_res(out, g_ref, be_ref, res):
    mu = jnp.mean(out, axis=0, keepdims=True)
    var = jnp.mean((out - mu) ** 2, axis=0, keepdims=True)
    y = g_ref[...] * (out - mu) / jnp.sqrt(var + 1e-5) + be_ref[...]
    y = jnp.where(y >= 0, y, 0.01 * y)
    return y + res


def _tc_combine1(p, xroot, bias, gamma, beta, x):
    def body(p_ref, r_ref, b_ref, g_ref, be_ref, x_ref, o_ref):
        out = p_ref[0] + p_ref[1] + r_ref[...] + b_ref[...]
        o_ref[...] = _bn_lrelu_res(out, g_ref, be_ref, x_ref[...])

    return pl.pallas_call(
        body, out_shape=jax.ShapeDtypeStruct(x.shape, jnp.float32)
    )(p, xroot, bias.reshape(1, -1), gamma.reshape(1, -1),
      beta.reshape(1, -1), x)


def _tc_combine2(p, ic2, x1, wl, bl, wr, gamma, beta):
    def body(p_ref, ic_ref, x1_ref, wl_ref, bl_ref, wr_ref, g_ref,
             be_ref, o_ref):
        agg = (p_ref[0] + p_ref[1]) * ic_ref[...]
        out = (jnp.dot(agg, wl_ref[...], preferred_element_type=jnp.float32)
               + bl_ref[...]
               + jnp.dot(x1_ref[...], wr_ref[...],
                         preferred_element_type=jnp.float32))
        o_ref[...] = _bn_lrelu_res(out, g_ref, be_ref, x1_ref[...])

    return pl.pallas_call(
        body, out_shape=jax.ShapeDtypeStruct(x1.shape, jnp.float32)
    )(p, ic2, x1, wl, bl.reshape(1, -1), wr, gamma.reshape(1, -1),
      beta.reshape(1, -1))


def _sc_mesh():
    return plsc.VectorSubcoreMesh(core_axis_name="c", subcore_axis_name="s")


_SC_PARAMS = pltpu.CompilerParams(use_tc_tiling_on_sc=False,
                                  needs_layout_passes=False)


def _striped(sid, total_rows, stripe, fn):
    nstripes = total_rows // stripe
    for t in range((nstripes + NS - 1) // NS):
        off = pl.multiple_of((sid + t * NS) * stripe, 8)
        if (t + 1) * NS <= nstripes:
            fn(off)
        else:
            @pl.when(sid + t * NS < nstripes)
            def _():
                fn(off)


def _sc_counts(widx, rn, e):
    kc = 400
    ew = e // NW
    ch = ew // kc
    zr = 1000
    nb = 3

    @functools.partial(
        pl.kernel,
        out_type=jax.ShapeDtypeStruct((NC * rn, CW), jnp.float32),
        mesh=_sc_mesh(),
        compiler_params=_SC_PARAMS,
        scratch_types=(
            [pltpu.VMEM((kc,), jnp.int32) for _ in range(nb)]
            + [pltpu.VMEM((kc, CW), jnp.float32),
               pltpu.VMEM((zr, CW), jnp.float32)]
            + [pltpu.SemaphoreType.DMA for _ in range(2 * nb)]
            + [pltpu.VMEM_SHARED((rn, CW), jnp.float32)]
        ),
    )
    def k(widx_hbm, out_hbm, i0, i1, i2, ones_v, zero_v,
          a0, a1, a2, s0, s1, s2, acc_sh):
        idxs = [i0, i1, i2]
        isem = [a0, a1, a2]
        ssem = [s0, s1, s2]
        cid = lax.axis_index("c")
        sid = lax.axis_index("s")
        tid = cid * NS + sid
        for j in range(zr):
            zero_v[j, :] = jnp.zeros((CW,), jnp.float32)
        for j in range(kc):
            ones_v[j, :] = jnp.ones((CW,), jnp.float32)
        _striped(sid, rn, zr,
                 lambda off: pltpu.sync_copy(zero_v,
                                             acc_sh.at[pl.ds(off, zr)]))
        plsc.subcore_barrier()

        def start(c, b):
            pltpu.async_copy(widx_hbm.at[pl.ds(tid * ew + c * kc, kc)],
                             idxs[b], isem[b])

        def finish(c, b):
            pltpu.make_async_copy(widx_hbm.at[pl.ds(tid * ew + c * kc, kc)],
                                  idxs[b], isem[b]).wait()
            pltpu.async_copy(ones_v, acc_sh.at[idxs[b]], ssem[b], add=True)

        def drain(b):
            pltpu.make_async_copy(ones_v, acc_sh.at[idxs[b]],
                                  ssem[b]).wait()

        start(0, 0)
        start(1, 1)

        def group(g, carry):
            for b in range(nb):
                c = g * nb + b

                @pl.when(c < ch)
                def _():
                    finish(c, b)
                    b2 = (b + 2) % nb

                    @pl.when((c + 2 < ch) & (c >= 1))
                    def _():
                        drain(b2)

                    @pl.when(c + 2 < ch)
                    def _():
                        start(c + 2, b2)
            return carry

        lax.fori_loop(0, (ch + nb - 1) // nb, group, 0)
        for last in range(ch - 3, ch):
            drain(last % nb)
        plsc.subcore_barrier()
        _striped(sid, rn, zr,
                 lambda off: pltpu.sync_copy(
                     acc_sh.at[pl.ds(off, zr)],
                     out_hbm.at[pl.ds(cid * rn + off, zr)]))

    return k(widx)


def _sc_rgcn(table, packed, invc_rep, n, d, e):
    ew = e // NW
    ch = ew // K
    zr = 40
    nb = 3

    @functools.partial(
        pl.kernel,
        out_type=jax.ShapeDtypeStruct((NC * n, d), jnp.float32),
        mesh=_sc_mesh(),
        compiler_params=_SC_PARAMS,
        scratch_types=(
            [pltpu.VMEM((3 * K,), jnp.int32) for _ in range(nb)]
            + [pltpu.VMEM((K, d), jnp.float32) for _ in range(nb)]
            + [pltpu.VMEM((K, CW), jnp.float32) for _ in range(nb)]
            + [pltpu.VMEM((zr, d), jnp.float32)]
            + [pltpu.SemaphoreType.DMA for _ in range(2 * nb)]
            + [pltpu.VMEM_SHARED((n, d), jnp.float32)]
        ),
    )
    def k(table_hbm, packed_hbm, invc_hbm, out_hbm,
          i0, i1, i2, r0, r1, r2, w0, w1, w2, zero_v,
          g0, g1, g2, s0, s1, s2, acc_sh):
        idx3 = [i0, i1, i2]
        rows = [r0, r1, r2]
        wc = [w0, w1, w2]
        gsem = [g0, g1, g2]
        ssem = [s0, s1, s2]
        cid = lax.axis_index("c")
        sid = lax.axis_index("s")
        tid = cid * NS + sid
        cbase = tid * ch

        for j in range(zr):
            for q in range(d // 16):
                zero_v[j, pl.ds(q * 16, 16)] = jnp.zeros((16,), jnp.float32)
        _striped(sid, n, zr,
                 lambda off: pltpu.sync_copy(zero_v,
                                             acc_sh.at[pl.ds(off, zr)]))
        plsc.subcore_barrier()

        def start(c, b):
            pltpu.sync_copy(
                packed_hbm.at[pl.ds((cbase + c) * 3 * K, 3 * K)], idx3[b])
            pltpu.async_copy(table_hbm.at[idx3[b].at[pl.ds(0, K)]],
                             rows[b], gsem[b])
            pltpu.async_copy(invc_hbm.at[idx3[b].at[pl.ds(2 * K, K)]],
                             wc[b], gsem[b])

        def finish(c, b):
            pltpu.make_async_copy(table_hbm.at[idx3[b].at[pl.ds(0, K)]],
                                  rows[b], gsem[b]).wait()
            pltpu.make_async_copy(invc_hbm.at[idx3[b].at[pl.ds(2 * K, K)]],
                                  wc[b], gsem[b]).wait()
            for ei in range(K):
                wspl = wc[b][ei, :]
                for q in range(d // 16):
                    sl = pl.ds(q * 16, 16)
                    rows[b][ei, sl] = rows[b][ei, sl] * wspl
            pltpu.async_copy(rows[b], acc_sh.at[idx3[b].at[pl.ds(K, K)]],
                             ssem[b], add=True)

        def drain(b):
            pltpu.make_async_copy(rows[b],
                                  acc_sh.at[idx3[b].at[pl.ds(K, K)]],
                                  ssem[b]).wait()

        start(0, 0)
        start(1, 1)

        def group(g, carry):
            for b in range(nb):
                c = g * nb + b

                @pl.when(c < ch)
                def _():
                    finish(c, b)
                    b2 = (b + 2) % nb

                    @pl.when((c + 2 < ch) & (c >= 1))
                    def _():
                        drain(b2)

                    @pl.when(c + 2 < ch)
                    def _():
                        start(c + 2, b2)
            return carry

        lax.fori_loop(0, (ch + nb - 1) // nb, group, 0)
        for last in range(ch - 3, ch):
            drain(last % nb)
        plsc.subcore_barrier()
        _striped(sid, n, zr,
                 lambda off: pltpu.sync_copy(
                     acc_sh.at[pl.ds(off, zr)],
                     out_hbm.at[pl.ds(cid * n + off, zr)]))

    return k(table, packed, invc_rep)


def _sc_sage(x1, packed2, n, d, e):
    ew = e // NW
    ch = ew // K
    zr = 40
    nb = 3

    @functools.partial(
        pl.kernel,
        out_type=jax.ShapeDtypeStruct((NC * n, d), jnp.float32),
        mesh=_sc_mesh(),
        compiler_params=_SC_PARAMS,
        scratch_types=(
            [pltpu.VMEM((2 * K,), jnp.int32) for _ in range(nb)]
            + [pltpu.VMEM((K, d), jnp.float32) for _ in range(nb)]
            + [pltpu.VMEM((zr, d), jnp.float32)]
            + [pltpu.SemaphoreType.DMA for _ in range(2 * nb)]
            + [pltpu.VMEM_SHARED((n, d), jnp.float32)]
        ),
    )
    def k(x1_hbm, packed_hbm, out_hbm,
          i0, i1, i2, r0, r1, r2, zero_v,
          g0, g1, g2, s0, s1, s2, acc_sh):
        idx2 = [i0, i1, i2]
        rows = [r0, r1, r2]
        gsem = [g0, g1, g2]
        ssem = [s0, s1, s2]
        cid = lax.axis_index("c")
        sid = lax.axis_index("s")
        tid = cid * NS + sid
        cbase = tid * ch

        for j in range(zr):
            for q in range(d // 16):
                zero_v[j, pl.ds(q * 16, 16)] = jnp.zeros((16,), jnp.float32)
        _striped(sid, n, zr,
                 lambda off: pltpu.sync_copy(zero_v,
                                             acc_sh.at[pl.ds(off, zr)]))
        plsc.subcore_barrier()

        def start(c, b):
            pltpu.sync_copy(
                packed_hbm.at[pl.ds((cbase + c) * 2 * K, 2 * K)], idx2[b])
            pltpu.async_copy(x1_hbm.at[idx2[b].at[pl.ds(0, K)]],
                             rows[b], gsem[b])

        def finish(c, b):
            pltpu.make_async_copy(x1_hbm.at[idx2[b].at[pl.ds(0, K)]],
                                  rows[b], gsem[b]).wait()
            pltpu.async_copy(rows[b], acc_sh.at[idx2[b].at[pl.ds(K, K)]],
                             ssem[b], add=True)

        def drain(b):
            pltpu.make_async_copy(rows[b],
                                  acc_sh.at[idx2[b].at[pl.ds(K, K)]],
                                  ssem[b]).wait()

        start(0, 0)
        start(1, 1)

        def group(g, carry):
            for b in range(nb):
                c = g * nb + b

                @pl.when(c < ch)
                def _():
                    finish(c, b)
                    b2 = (b + 2) % nb

                    @pl.when((c + 2 < ch) & (c >= 1))
                    def _():
                        drain(b2)

                    @pl.when(c + 2 < ch)
                    def _():
                        start(c + 2, b2)
            return carry

        lax.fori_loop(0, (ch + nb - 1) // nb, group, 0)
        for last in range(ch - 3, ch):
            drain(last % nb)
        plsc.subcore_barrier()
        _striped(sid, n, zr,
                 lambda off: pltpu.sync_copy(
                     acc_sh.at[pl.ds(off, zr)],
                     out_hbm.at[pl.ds(cid * n + off, zr)]))

    return k(x1, packed2)


def kernel(node_features, node_type, edge_index, edge_type, rgcn_weight,
           rgcn_root, rgcn_bias, bn1_gamma, bn1_beta, sage_lin_l_w,
           sage_lin_l_b, sage_lin_r_w, bn3_gamma, bn3_beta):
    del node_type
    n, d = node_features.shape
    e = edge_type.shape[0]
    r = rgcn_weight.shape[0]
    x = node_features
    src = edge_index[0]
    dst = edge_index[1]

    wstack = jnp.concatenate([rgcn_weight, rgcn_root[None]], axis=0)
    xw = _tc_matmul_stacked(x, wstack)
    table = xw.reshape((r + 1) * n, d)
    xroot = xw[r]

    e2 = (e // 128, 128)
    gidx2, widx2 = _tc_index_prep(src.reshape(e2), dst.reshape(e2),
                                  edge_type.reshape(e2), n)
    gidx = gidx2.reshape(e)
    widx = widx2.reshape(e)

    cpart = _sc_counts(widx, r * n, e)
    counts = cpart[:, 0].reshape(2, r, n)
    invc, invc2 = _tc_inv(counts)

    invc_rep = jnp.broadcast_to(invc.reshape(r * n)[:, None], (r * n, CW))
    nck = e // K
    packed3 = jnp.stack([gidx.reshape(nck, K), dst.reshape(nck, K),
                         widx.reshape(nck, K)], axis=1).reshape(3 * e)
    p1 = _sc_rgcn(table, packed3, invc_rep, n, d, e)
    p1 = p1.reshape(2, n, d)
    x1 = _tc_combine1(p1, xroot, rgcn_bias, bn1_gamma, bn1_beta, x)

    packed2 = jnp.stack([src.reshape(nck, K), dst.reshape(nck, K)],
                        axis=1).reshape(2 * e)
    p2 = _sc_sage(x1, packed2, n, d, e).reshape(2, n, d)
    x3 = _tc_combine2(p2, invc2, x1, sage_lin_l_w, sage_lin_l_b,
                      sage_lin_r_w, bn3_gamma, bn3_beta)
    return x3


# speedup vs baseline: 1.1034x; 1.1034x over previous
import functools

import jax
import jax.numpy as jnp
from jax import lax
from jax.experimental import pallas as pl
from jax.experimental.pallas import tpu as pltpu
from jax.experimental.pallas import tpu_sc as plsc

NC = 2
NS = 16
NW = NC * NS
K = 80
CW = 16


def _tc_matmul_prep(x, wstack, src2, dst2, et2, n):
    g, d, _ = wstack.shape
    blk = 400
    nb = n // blk
    def body(x_ref, w_ref, s_ref, d_ref, t_ref, o_ref, g_ref, wi_ref):
        o_ref[0] = jnp.dot(x_ref[...], w_ref[0],
                           preferred_element_type=jnp.float32)
        t = t_ref[...]
        g_ref[...] = t * n + s_ref[...]
        wi_ref[...] = t * n + d_ref[...]

    eshp = jax.ShapeDtypeStruct(src2.shape, jnp.int32)
    espec = pl.BlockSpec((1, 1, src2.shape[2]),
                         lambda r, i: (r * nb + i, 0, 0))
    return pl.pallas_call(
        body,
        grid=(g, nb),
        in_specs=[
            pl.BlockSpec((blk, d), lambda r, i: (i, 0)),
            pl.BlockSpec((1, d, d), lambda r, i: (r, 0, 0)),
            espec, espec, espec,
        ],
        out_specs=(pl.BlockSpec((1, blk, d), lambda r, i: (r, i, 0)),
                   espec, espec),
        out_shape=(jax.ShapeDtypeStruct((g, n, d), jnp.float32),
                   eshp, eshp),
    )(x, wstack, src2, dst2, et2)


def _tc_inv(cpart, rn, n):
    rn2 = rn * CW // 128
    n2 = n * CW // 128

    def body(c_ref, ic_ref, ic2_ref):
        cs = c_ref[pl.ds(0, rn2), :] + c_ref[pl.ds(rn2, rn2), :]
        ic_ref[...] = 1.0 / jnp.maximum(cs, 1.0)
        c2 = sum(cs[q * n2:(q + 1) * n2, :] for q in range(rn // n))
        ic2_ref[...] = 1.0 / jnp.maximum(c2, 1.0)

    ic, ic2 = pl.pallas_call(
        body,
        out_shape=(jax.ShapeDtypeStruct((rn2, 128), jnp.float32),
                   jax.ShapeDtypeStruct((n2, 128), jnp.float32)),
    )(cpart.reshape(2 * rn2, 128))
    return ic.reshape(rn, CW), ic2.reshape(n, CW)


def _bn_lrelu_res(out, g_ref, be_ref, res):
    mu = jnp.mean(out, axis=0, keepdims=True)
    var = jnp.mean((out - mu) ** 2, axis=0, keepdims=True)
    y = g_ref[...] * (out - mu) / jnp.sqrt(var + 1e-5) + be_ref[...]
    y = jnp.where(y >= 0, y, 0.01 * y)
    return y + res


def _tc_combine1(p, xroot, bias, gamma, beta, x):
    def body(p_ref, r_ref, b_ref, g_ref, be_ref, x_ref, o_ref):
        out = p_ref[0] + p_ref[1] + r_ref[...] + b_ref[...]
        o_ref[...] = _bn_lrelu_res(out, g_ref, be_ref, x_ref[...])

    return pl.pallas_call(
        body, out_shape=jax.ShapeDtypeStruct(x.shape, jnp.float32)
    )(p, xroot, bias.reshape(1, -1), gamma.reshape(1, -1),
      beta.reshape(1, -1), x)


def _tc_combine2(p, ic2, x1, wl, bl, wr, gamma, beta):
    def body(p_ref, ic_ref, x1_ref, wl_ref, bl_ref, wr_ref, g_ref,
             be_ref, o_ref):
        agg = (p_ref[0] + p_ref[1]) * ic_ref[:, 0:1]
        out = (jnp.dot(agg, wl_ref[...], preferred_element_type=jnp.float32)
               + bl_ref[...]
               + jnp.dot(x1_ref[...], wr_ref[...],
                         preferred_element_type=jnp.float32))
        o_ref[...] = _bn_lrelu_res(out, g_ref, be_ref, x1_ref[...])

    return pl.pallas_call(
        body, out_shape=jax.ShapeDtypeStruct(x1.shape, jnp.float32)
    )(p, ic2, x1, wl, bl.reshape(1, -1), wr, gamma.reshape(1, -1),
      beta.reshape(1, -1))


def _sc_mesh():
    return plsc.VectorSubcoreMesh(core_axis_name="c", subcore_axis_name="s")


_SC_PARAMS = pltpu.CompilerParams(use_tc_tiling_on_sc=False,
                                  needs_layout_passes=False)


def _striped(sid, total_rows, stripe, fn):
    nstripes = total_rows // stripe
    for t in range((nstripes + NS - 1) // NS):
        off = pl.multiple_of((sid + t * NS) * stripe, 8)
        if (t + 1) * NS <= nstripes:
            fn(off)
        else:
            @pl.when(sid + t * NS < nstripes)
            def _():
                fn(off)


def _sc_counts(widx, rn, e):
    kc = 400
    ew = e // NW
    ch = ew // kc
    zr = 1000
    nb = 3

    @functools.partial(
        pl.kernel,
        out_type=jax.ShapeDtypeStruct((NC * rn, CW), jnp.float32),
        mesh=_sc_mesh(),
        compiler_params=_SC_PARAMS,
        scratch_types=(
            [pltpu.VMEM((kc,), jnp.int32) for _ in range(nb)]
            + [pltpu.VMEM((kc, CW), jnp.float32),
               pltpu.VMEM((zr, CW), jnp.float32)]
            + [pltpu.SemaphoreType.DMA for _ in range(2 * nb)]
            + [pltpu.VMEM_SHARED((rn, CW), jnp.float32)]
        ),
    )
    def k(widx_hbm, out_hbm, i0, i1, i2, ones_v, zero_v,
          a0, a1, a2, s0, s1, s2, acc_sh):
        idxs = [i0, i1, i2]
        isem = [a0, a1, a2]
        ssem = [s0, s1, s2]
        cid = lax.axis_index("c")
        sid = lax.axis_index("s")
        tid = cid * NS + sid
        for j in range(zr):
            zero_v[j, :] = jnp.zeros((CW,), jnp.float32)
        for j in range(kc):
            ones_v[j, :] = jnp.ones((CW,), jnp.float32)
        _striped(sid, rn, zr,
                 lambda off: pltpu.sync_copy(zero_v,
                                             acc_sh.at[pl.ds(off, zr)]))
        plsc.subcore_barrier()

        def start(c, b):
            pltpu.async_copy(widx_hbm.at[pl.ds(tid * ew + c * kc, kc)],
                             idxs[b], isem[b])

        def finish(c, b):
            pltpu.make_async_copy(widx_hbm.at[pl.ds(tid * ew + c * kc, kc)],
                                  idxs[b], isem[b]).wait()
            pltpu.async_copy(ones_v, acc_sh.at[idxs[b]], ssem[b], add=True)

        def drain(b):
            pltpu.make_async_copy(ones_v, acc_sh.at[idxs[b]],
                                  ssem[b]).wait()

        start(0, 0)
        start(1, 1)

        def group(g, carry):
            for b in range(nb):
                c = g * nb + b

                @pl.when(c < ch)
                def _():
                    finish(c, b)
                    b2 = (b + 2) % nb

                    @pl.when((c + 2 < ch) & (c >= 1))
                    def _():
                        drain(b2)

                    @pl.when(c + 2 < ch)
                    def _():
                        start(c + 2, b2)
            return carry

        lax.fori_loop(0, (ch + nb - 1) // nb, group, 0)
        for last in range(ch - 3, ch):
            drain(last % nb)
        plsc.subcore_barrier()
        _striped(sid, rn, zr,
                 lambda off: pltpu.sync_copy(
                     acc_sh.at[pl.ds(off, zr)],
                     out_hbm.at[pl.ds(cid * rn + off, zr)]))

    return k(widx)


def _sc_rgcn(table, packed, invc_rep, n, d, e):
    ew = e // NW
    ch = ew // K
    zr = 40
    nb = 3

    @functools.partial(
        pl.kernel,
        out_type=jax.ShapeDtypeStruct((NC * n, d), jnp.float32),
        mesh=_sc_mesh(),
        compiler_params=_SC_PARAMS,
        scratch_types=(
            [pltpu.VMEM((3 * K,), jnp.int32) for _ in range(nb)]
            + [pltpu.VMEM((K, d), jnp.float32) for _ in range(nb)]
            + [pltpu.VMEM((K, CW), jnp.float32) for _ in range(nb)]
            + [pltpu.VMEM((zr, d), jnp.float32)]
            + [pltpu.SemaphoreType.DMA for _ in range(2 * nb)]
            + [pltpu.VMEM_SHARED((n, d), jnp.float32)]
        ),
    )
    def k(table_hbm, packed_hbm, invc_hbm, out_hbm,
          i0, i1, i2, r0, r1, r2, w0, w1, w2, zero_v,
          g0, g1, g2, s0, s1, s2, acc_sh):
        idx3 = [i0, i1, i2]
        rows = [r0, r1, r2]
        wc = [w0, w1, w2]
        gsem = [g0, g1, g2]
        ssem = [s0, s1, s2]
        cid = lax.axis_index("c")
        sid = lax.axis_index("s")
        tid = cid * NS + sid
        cbase = tid * ch

        for j in range(zr):
            for q in range(d // 16):
                zero_v[j, pl.ds(q * 16, 16)] = jnp.zeros((16,), jnp.float32)
        _striped(sid, n, zr,
                 lambda off: pltpu.sync_copy(zero_v,
                                             acc_sh.at[pl.ds(off, zr)]))
        plsc.subcore_barrier()

        def start(c, b):
            pltpu.sync_copy(
                packed_hbm.at[pl.ds((cbase + c) * 3 * K, 3 * K)], idx3[b])
            pltpu.async_copy(table_hbm.at[idx3[b].at[pl.ds(0, K)]],
                             rows[b], gsem[b])
            pltpu.async_copy(invc_hbm.at[idx3[b].at[pl.ds(2 * K, K)]],
                             wc[b], gsem[b])

        def finish(c, b):
            pltpu.make_async_copy(table_hbm.at[idx3[b].at[pl.ds(0, K)]],
                                  rows[b], gsem[b]).wait()
            pltpu.make_async_copy(invc_hbm.at[idx3[b].at[pl.ds(2 * K, K)]],
                                  wc[b], gsem[b]).wait()
            for ei in range(K):
                wspl = wc[b][ei, :]
                for q in range(d // 16):
                    sl = pl.ds(q * 16, 16)
                    rows[b][ei, sl] = rows[b][ei, sl] * wspl
            pltpu.async_copy(rows[b], acc_sh.at[idx3[b].at[pl.ds(K, K)]],
                             ssem[b], add=True)

        def drain(b):
            pltpu.make_async_copy(rows[b],
                                  acc_sh.at[idx3[b].at[pl.ds(K, K)]],
                                  ssem[b]).wait()

        start(0, 0)
        start(1, 1)

        def group(g, carry):
            for b in range(nb):
                c = g * nb + b

                @pl.when(c < ch)
                def _():
                    finish(c, b)
                    b2 = (b + 2) % nb

                    @pl.when((c + 2 < ch) & (c >= 1))
                    def _():
                        drain(b2)

                    @pl.when(c + 2 < ch)
                    def _():
                        start(c + 2, b2)
            return carry

        lax.fori_loop(0, (ch + nb - 1) // nb, group, 0)
        for last in range(ch - 3, ch):
            drain(last % nb)
        plsc.subcore_barrier()
        _striped(sid, n, zr,
                 lambda off: pltpu.sync_copy(
                     acc_sh.at[pl.ds(off, zr)],
                     out_hbm.at[pl.ds(cid * n + off, zr)]))

    return k(table, packed, invc_rep)


def _sc_sage(x1, packed2, n, d, e):
    ew = e // NW
    ch = ew // K
    zr = 40
    nb = 3

    @functools.partial(
        pl.kernel,
        out_type=jax.ShapeDtypeStruct((NC * n, d), jnp.float32),
        mesh=_sc_mesh(),
        compiler_params=_SC_PARAMS,
        scratch_types=(
            [pltpu.VMEM((2 * K,), jnp.int32) for _ in range(nb)]
            + [pltpu.VMEM((K, d), jnp.float32) for _ in range(nb)]
            + [pltpu.VMEM((zr, d), jnp.float32)]
            + [pltpu.SemaphoreType.DMA for _ in range(2 * nb)]
            + [pltpu.VMEM_SHARED((n, d), jnp.float32)]
        ),
    )
    def k(x1_hbm, packed_hbm, out_hbm,
          i0, i1, i2, r0, r1, r2, zero_v,
          g0, g1, g2, s0, s1, s2, acc_sh):
        idx2 = [i0, i1, i2]
        rows = [r0, r1, r2]
        gsem = [g0, g1, g2]
        ssem = [s0, s1, s2]
        cid = lax.axis_index("c")
        sid = lax.axis_index("s")
        tid = cid * NS + sid
        cbase = tid * ch

        for j in range(zr):
            for q in range(d // 16):
                zero_v[j, pl.ds(q * 16, 16)] = jnp.zeros((16,), jnp.float32)
        _striped(sid, n, zr,
                 lambda off: pltpu.sync_copy(zero_v,
                                             acc_sh.at[pl.ds(off, zr)]))
        plsc.subcore_barrier()

        def start(c, b):
            pltpu.sync_copy(
                packed_hbm.at[pl.ds((cbase + c) * 2 * K, 2 * K)], idx2[b])
            pltpu.async_copy(x1_hbm.at[idx2[b].at[pl.ds(0, K)]],
                             rows[b], gsem[b])

        def finish(c, b):
            pltpu.make_async_copy(x1_hbm.at[idx2[b].at[pl.ds(0, K)]],
                                  rows[b], gsem[b]).wait()
            pltpu.async_copy(rows[b], acc_sh.at[idx2[b].at[pl.ds(K, K)]],
                             ssem[b], add=True)

        def drain(b):
            pltpu.make_async_copy(rows[b],
                                  acc_sh.at[idx2[b].at[pl.ds(K, K)]],
                                  ssem[b]).wait()

        start(0, 0)
        start(1, 1)

        def group(g, carry):
            for b in range(nb):
                c = g * nb + b

                @pl.when(c < ch)
                def _():
                    finish(c, b)
                    b2 = (b + 2) % nb

                    @pl.when((c + 2 < ch) & (c >= 1))
                    def _():
                        drain(b2)

                    @pl.when(c + 2 < ch)
                    def _():
                        start(c + 2, b2)
            return carry

        lax.fori_loop(0, (ch + nb - 1) // nb, group, 0)
        for last in range(ch - 3, ch):
            drain(last % nb)
        plsc.subcore_barrier()
        _striped(sid, n, zr,
                 lambda off: pltpu.sync_copy(
                     acc_sh.at[pl.ds(off, zr)],
                     out_hbm.at[pl.ds(cid * n + off, zr)]))

    return k(x1, packed2)


def kernel(node_features, node_type, edge_index, edge_type, rgcn_weight,
           rgcn_root, rgcn_bias, bn1_gamma, bn1_beta, sage_lin_l_w,
           sage_lin_l_b, sage_lin_r_w, bn3_gamma, bn3_beta):
    del node_type
    n, d = node_features.shape
    e = edge_type.shape[0]
    r = rgcn_weight.shape[0]
    x = node_features
    src = edge_index[0]
    dst = edge_index[1]

    wstack = jnp.concatenate([rgcn_weight, rgcn_root[None]], axis=0)
    nb = (r + 1) * (n // 400)
    e2 = (nb, 1, e // nb)
    xw, gidx2, widx2 = _tc_matmul_prep(x, wstack, src.reshape(e2),
                                       dst.reshape(e2),
                                       edge_type.reshape(e2), n)
    table = xw.reshape((r + 1) * n, d)
    xroot = xw[r]
    gidx = gidx2.reshape(e)
    widx = widx2.reshape(e)

    cpart = _sc_counts(widx, r * n, e)
    invc_rep, invc2 = _tc_inv(cpart, r * n, n)

    nck = e // K
    packed3 = jnp.stack([gidx.reshape(nck, K), dst.reshape(nck, K),
                         widx.reshape(nck, K)], axis=1).reshape(3 * e)
    p1 = _sc_rgcn(table, packed3, invc_rep, n, d, e)
    p1 = p1.reshape(2, n, d)
    x1 = _tc_combine1(p1, xroot, rgcn_bias, bn1_gamma, bn1_beta, x)

    packed2 = jnp.stack([src.reshape(nck, K), dst.reshape(nck, K)],
                        axis=1).reshape(2 * e)
    p2 = _sc_sage(x1, packed2, n, d, e).reshape(2, n, d)
    x3 = _tc_combine2(p2, invc2, x1, sage_lin_l_w, sage_lin_l_b,
                      sage_lin_r_w, bn3_gamma, bn3_beta)
    return x3


# speedup vs baseline: 1.1478x; 1.0402x over previous
import functools

import jax
import jax.numpy as jnp
from jax import lax
from jax.experimental import pallas as pl
from jax.experimental.pallas import tpu as pltpu
from jax.experimental.pallas import tpu_sc as plsc

NC = 2
NS = 16
NW = NC * NS
K = 80
CW = 16


def _tc_matmul_prep(x, wstack, src2, dst2, et2, n):
    g, d, _ = wstack.shape
    blk = 400
    nb = n // blk
    def body(x_ref, w_ref, s_ref, d_ref, t_ref, o_ref, g_ref, wi_ref):
        o_ref[0] = jnp.dot(x_ref[...], w_ref[0],
                           preferred_element_type=jnp.float32)
        t = t_ref[...]
        g_ref[...] = t * n + s_ref[...]
        wi_ref[...] = t * n + d_ref[...]

    eshp = jax.ShapeDtypeStruct(src2.shape, jnp.int32)
    espec = pl.BlockSpec((1, 1, src2.shape[2]),
                         lambda r, i: (r * nb + i, 0, 0))
    return pl.pallas_call(
        body,
        grid=(g, nb),
        in_specs=[
            pl.BlockSpec((blk, d), lambda r, i: (i, 0)),
            pl.BlockSpec((1, d, d), lambda r, i: (r, 0, 0)),
            espec, espec, espec,
        ],
        out_specs=(pl.BlockSpec((1, blk, d), lambda r, i: (r, i, 0)),
                   espec, espec),
        out_shape=(jax.ShapeDtypeStruct((g, n, d), jnp.float32),
                   eshp, eshp),
    )(x, wstack, src2, dst2, et2)


def _tc_inv(cpart, rn, n):
    rn2 = rn * CW // 128
    n2 = n * CW // 128

    def body(c_ref, ic_ref, ic2_ref):
        cs = c_ref[pl.ds(0, rn2), :] + c_ref[pl.ds(rn2, rn2), :]
        ic_ref[...] = 1.0 / jnp.maximum(cs, 1.0)
        c2 = sum(cs[q * n2:(q + 1) * n2, :] for q in range(rn // n))
        ic2_ref[...] = 1.0 / jnp.maximum(c2, 1.0)

    ic, ic2 = pl.pallas_call(
        body,
        out_shape=(jax.ShapeDtypeStruct((rn2, 128), jnp.float32),
                   jax.ShapeDtypeStruct((n2, 128), jnp.float32)),
    )(cpart.reshape(2 * rn2, 128))
    return ic.reshape(rn, CW), ic2.reshape(n, CW)


def _bn_lrelu_res(out, g_ref, be_ref, res):
    mu = jnp.mean(out, axis=0, keepdims=True)
    var = jnp.mean((out - mu) ** 2, axis=0, keepdims=True)
    y = g_ref[...] * (out - mu) / jnp.sqrt(var + 1e-5) + be_ref[...]
    y = jnp.where(y >= 0, y, 0.01 * y)
    return y + res


def _tc_combine1(p, xroot, bias, gamma, beta, x):
    def body(p_ref, r_ref, b_ref, g_ref, be_ref, x_ref, o_ref):
        out = p_ref[0] + p_ref[1] + r_ref[...] + b_ref[...]
        o_ref[...] = _bn_lrelu_res(out, g_ref, be_ref, x_ref[...])

    return pl.pallas_call(
        body, out_shape=jax.ShapeDtypeStruct(x.shape, jnp.float32)
    )(p, xroot, bias.reshape(1, -1), gamma.reshape(1, -1),
      beta.reshape(1, -1), x)


def _tc_combine2(p, ic2, x1, wl, bl, wr, gamma, beta):
    def body(p_ref, ic_ref, x1_ref, wl_ref, bl_ref, wr_ref, g_ref,
             be_ref, o_ref):
        agg = (p_ref[0] + p_ref[1]) * ic_ref[:, 0:1]
        out = (jnp.dot(agg, wl_ref[...], preferred_element_type=jnp.float32)
               + bl_ref[...]
               + jnp.dot(x1_ref[...], wr_ref[...],
                         preferred_element_type=jnp.float32))
        o_ref[...] = _bn_lrelu_res(out, g_ref, be_ref, x1_ref[...])

    return pl.pallas_call(
        body, out_shape=jax.ShapeDtypeStruct(x1.shape, jnp.float32)
    )(p, ic2, x1, wl, bl.reshape(1, -1), wr, gamma.reshape(1, -1),
      beta.reshape(1, -1))


def _sc_mesh():
    return plsc.VectorSubcoreMesh(core_axis_name="c", subcore_axis_name="s")


_SC_PARAMS = pltpu.CompilerParams(use_tc_tiling_on_sc=False,
                                  needs_layout_passes=False)


def _striped(sid, total_rows, stripe, fn):
    nstripes = total_rows // stripe
    for t in range((nstripes + NS - 1) // NS):
        off = pl.multiple_of((sid + t * NS) * stripe, 8)
        if (t + 1) * NS <= nstripes:
            fn(off)
        else:
            @pl.when(sid + t * NS < nstripes)
            def _():
                fn(off)


def _sc_counts(widx, rn, e):
    kc = 400
    ew = e // NW
    ch = ew // kc
    zr = 1000
    nb = 3

    @functools.partial(
        pl.kernel,
        out_type=jax.ShapeDtypeStruct((NC * rn, CW), jnp.float32),
        mesh=_sc_mesh(),
        compiler_params=_SC_PARAMS,
        scratch_types=(
            [pltpu.VMEM((kc,), jnp.int32) for _ in range(nb)]
            + [pltpu.VMEM((kc, CW), jnp.float32),
               pltpu.VMEM((zr, CW), jnp.float32)]
            + [pltpu.SemaphoreType.DMA for _ in range(2 * nb)]
            + [pltpu.VMEM_SHARED((rn, CW), jnp.float32)]
        ),
    )
    def k(widx_hbm, out_hbm, i0, i1, i2, ones_v, zero_v,
          a0, a1, a2, s0, s1, s2, acc_sh):
        idxs = [i0, i1, i2]
        isem = [a0, a1, a2]
        ssem = [s0, s1, s2]
        cid = lax.axis_index("c")
        sid = lax.axis_index("s")
        tid = cid * NS + sid
        for j in range(zr):
            zero_v[j, :] = jnp.zeros((CW,), jnp.float32)
        for j in range(kc):
            ones_v[j, :] = jnp.ones((CW,), jnp.float32)
        _striped(sid, rn, zr,
                 lambda off: pltpu.sync_copy(zero_v,
                                             acc_sh.at[pl.ds(off, zr)]))
        plsc.subcore_barrier()

        def start(c, b):
            pltpu.async_copy(widx_hbm.at[pl.ds(tid * ew + c * kc, kc)],
                             idxs[b], isem[b])

        def finish(c, b):
            pltpu.make_async_copy(widx_hbm.at[pl.ds(tid * ew + c * kc, kc)],
                                  idxs[b], isem[b]).wait()
            pltpu.async_copy(ones_v, acc_sh.at[idxs[b]], ssem[b], add=True)

        def drain(b):
            pltpu.make_async_copy(ones_v, acc_sh.at[idxs[b]],
                                  ssem[b]).wait()

        start(0, 0)
        start(1, 1)

        def group(g, carry):
            for b in range(nb):
                c = g * nb + b

                @pl.when(c < ch)
                def _():
                    finish(c, b)
                    b2 = (b + 2) % nb

                    @pl.when((c + 2 < ch) & (c >= 1))
                    def _():
                        drain(b2)

                    @pl.when(c + 2 < ch)
                    def _():
                        start(c + 2, b2)
            return carry

        lax.fori_loop(0, (ch + nb - 1) // nb, group, 0)
        for last in range(ch - 3, ch):
            drain(last % nb)
        plsc.subcore_barrier()
        _striped(sid, rn, zr,
                 lambda off: pltpu.sync_copy(
                     acc_sh.at[pl.ds(off, zr)],
                     out_hbm.at[pl.ds(cid * rn + off, zr)]))

    return k(widx)


def _sc_rgcn(table, packed, invc_rep, n, d, e):
    ew = e // NW
    ch = ew // K
    nb = 4

    @functools.partial(
        pl.kernel,
        out_type=jax.ShapeDtypeStruct((NC * n, d), jnp.float32),
        mesh=_sc_mesh(),
        compiler_params=_SC_PARAMS,
        scratch_types=(
            [pltpu.VMEM((3 * K,), jnp.int32) for _ in range(nb)]
            + [pltpu.VMEM((K, d), jnp.float32) for _ in range(nb)]
            + [pltpu.VMEM((K, CW), jnp.float32) for _ in range(nb)]
            + [pltpu.SemaphoreType.DMA for _ in range(2 * nb)]
            + [pltpu.VMEM_SHARED((n, d), jnp.float32)]
        ),
    )
    def k(table_hbm, packed_hbm, invc_hbm, out_hbm,
          i0, i1, i2, i3, r0, r1, r2, r3, w0, w1, w2, w3,
          g0, g1, g2, g3, s0, s1, s2, s3, acc_sh):
        idx3 = [i0, i1, i2, i3]
        rows = [r0, r1, r2, r3]
        wc = [w0, w1, w2, w3]
        gsem = [g0, g1, g2, g3]
        ssem = [s0, s1, s2, s3]
        cid = lax.axis_index("c")
        sid = lax.axis_index("s")
        tid = cid * NS + sid
        cbase = tid * ch

        for j in range(K):
            for q in range(d // 16):
                rows[0][j, pl.ds(q * 16, 16)] = jnp.zeros((16,), jnp.float32)
        _striped(sid, n, K,
                 lambda off: pltpu.sync_copy(rows[0],
                                             acc_sh.at[pl.ds(off, K)]))
        plsc.subcore_barrier()

        def start(c, b):
            pltpu.sync_copy(
                packed_hbm.at[pl.ds((cbase + c) * 3 * K, 3 * K)], idx3[b])
            pltpu.async_copy(table_hbm.at[idx3[b].at[pl.ds(0, K)]],
                             rows[b], gsem[b])
            pltpu.async_copy(invc_hbm.at[idx3[b].at[pl.ds(2 * K, K)]],
                             wc[b], gsem[b])

        def finish(c, b):
            pltpu.make_async_copy(table_hbm.at[idx3[b].at[pl.ds(0, K)]],
                                  rows[b], gsem[b]).wait()
            pltpu.make_async_copy(invc_hbm.at[idx3[b].at[pl.ds(2 * K, K)]],
                                  wc[b], gsem[b]).wait()
            for ei in range(K):
                wspl = wc[b][ei, :]
                for q in range(d // 16):
                    sl = pl.ds(q * 16, 16)
                    rows[b][ei, sl] = rows[b][ei, sl] * wspl
            pltpu.async_copy(rows[b], acc_sh.at[idx3[b].at[pl.ds(K, K)]],
                             ssem[b], add=True)

        def drain(b):
            pltpu.make_async_copy(rows[b],
                                  acc_sh.at[idx3[b].at[pl.ds(K, K)]],
                                  ssem[b]).wait()

        start(0, 0)
        start(1, 1)
        start(2, 2)

        def group(g, carry):
            for b in range(nb):
                c = g * nb + b

                @pl.when(c < ch)
                def _():
                    finish(c, b)
                    b2 = (b + 3) % nb

                    @pl.when((c + 3 < ch) & (c >= 1))
                    def _():
                        drain(b2)

                    @pl.when(c + 3 < ch)
                    def _():
                        start(c + 3, b2)
            return carry

        lax.fori_loop(0, (ch + nb - 1) // nb, group, 0)
        for last in range(ch - 4, ch):
            drain(last % nb)
        plsc.subcore_barrier()
        _striped(sid, n, K,
                 lambda off: pltpu.sync_copy(
                     acc_sh.at[pl.ds(off, K)],
                     out_hbm.at[pl.ds(cid * n + off, K)]))

    return k(table, packed, invc_rep)


def _sc_sage(x1, packed2, n, d, e):
    ew = e // NW
    ch = ew // K
    nb = 4

    @functools.partial(
        pl.kernel,
        out_type=jax.ShapeDtypeStruct((NC * n, d), jnp.float32),
        mesh=_sc_mesh(),
        compiler_params=_SC_PARAMS,
        scratch_types=(
            [pltpu.VMEM((2 * K,), jnp.int32) for _ in range(nb)]
            + [pltpu.VMEM((K, d), jnp.float32) for _ in range(nb)]
            + [pltpu.SemaphoreType.DMA for _ in range(2 * nb)]
            + [pltpu.VMEM_SHARED((n, d), jnp.float32)]
        ),
    )
    def k(x1_hbm, packed_hbm, out_hbm,
          i0, i1, i2, i3, r0, r1, r2, r3,
          g0, g1, g2, g3, s0, s1, s2, s3, acc_sh):
        idx2 = [i0, i1, i2, i3]
        rows = [r0, r1, r2, r3]
        gsem = [g0, g1, g2, g3]
        ssem = [s0, s1, s2, s3]
        cid = lax.axis_index("c")
        sid = lax.axis_index("s")
        tid = cid * NS + sid
        cbase = tid * ch

        for j in range(K):
            for q in range(d // 16):
                rows[0][j, pl.ds(q * 16, 16)] = jnp.zeros((16,), jnp.float32)
        _striped(sid, n, K,
                 lambda off: pltpu.sync_copy(rows[0],
                                             acc_sh.at[pl.ds(off, K)]))
        plsc.subcore_barrier()

        def start(c, b):
            pltpu.sync_copy(
                packed_hbm.at[pl.ds((cbase + c) * 2 * K, 2 * K)], idx2[b])
            pltpu.async_copy(x1_hbm.at[idx2[b].at[pl.ds(0, K)]],
                             rows[b], gsem[b])

        def finish(c, b):
            pltpu.make_async_copy(x1_hbm.at[idx2[b].at[pl.ds(0, K)]],
                                  rows[b], gsem[b]).wait()
            pltpu.async_copy(rows[b], acc_sh.at[idx2[b].at[pl.ds(K, K)]],
                             ssem[b], add=True)

        def drain(b):
            pltpu.make_async_copy(rows[b],
                                  acc_sh.at[idx2[b].at[pl.ds(K, K)]],
                                  ssem[b]).wait()

        start(0, 0)
        start(1, 1)
        start(2, 2)

        def group(g, carry):
            for b in range(nb):
                c = g * nb + b

                @pl.when(c < ch)
                def _():
                    finish(c, b)
                    b2 = (b + 3) % nb

                    @pl.when((c + 3 < ch) & (c >= 1))
                    def _():
                        drain(b2)

                    @pl.when(c + 3 < ch)
                    def _():
                        start(c + 3, b2)
            return carry

        lax.fori_loop(0, (ch + nb - 1) // nb, group, 0)
        for last in range(ch - 4, ch):
            drain(last % nb)
        plsc.subcore_barrier()
        _striped(sid, n, K,
                 lambda off: pltpu.sync_copy(
                     acc_sh.at[pl.ds(off, K)],
                     out_hbm.at[pl.ds(cid * n + off, K)]))

    return k(x1, packed2)


def kernel(node_features, node_type, edge_index, edge_type, rgcn_weight,
           rgcn_root, rgcn_bias, bn1_gamma, bn1_beta, sage_lin_l_w,
           sage_lin_l_b, sage_lin_r_w, bn3_gamma, bn3_beta):
    del node_type
    n, d = node_features.shape
    e = edge_type.shape[0]
    r = rgcn_weight.shape[0]
    x = node_features
    src = edge_index[0]
    dst = edge_index[1]

    wstack = jnp.concatenate([rgcn_weight, rgcn_root[None]], axis=0)
    nb = (r + 1) * (n // 400)
    e2 = (nb, 1, e // nb)
    xw, gidx2, widx2 = _tc_matmul_prep(x, wstack, src.reshape(e2),
                                       dst.reshape(e2),
                                       edge_type.reshape(e2), n)
    table = xw.reshape((r + 1) * n, d)
    xroot = xw[r]
    gidx = gidx2.reshape(e)
    widx = widx2.reshape(e)

    cpart = _sc_counts(widx, r * n, e)
    invc_rep, invc2 = _tc_inv(cpart, r * n, n)

    nck = e // K
    packed3 = jnp.stack([gidx.reshape(nck, K), dst.reshape(nck, K),
                         widx.reshape(nck, K)], axis=1).reshape(3 * e)
    p1 = _sc_rgcn(table, packed3, invc_rep, n, d, e)
    p1 = p1.reshape(2, n, d)
    x1 = _tc_combine1(p1, xroot, rgcn_bias, bn1_gamma, bn1_beta, x)

    packed2 = jnp.stack([src.reshape(nck, K), dst.reshape(nck, K)],
                        axis=1).reshape(2 * e)
    p2 = _sc_sage(x1, packed2, n, d, e).reshape(2, n, d)
    x3 = _tc_combine2(p2, invc2, x1, sage_lin_l_w, sage_lin_l_b,
                      sage_lin_r_w, bn3_gamma, bn3_beta)
    return x3
